# baseline (device time: 11073 ns/iter reference)
import jax
import jax.numpy as jnp
from jax import lax
from jax.experimental import pallas as pl
from jax.experimental.pallas import tpu as pltpu

N_DEV = 4
C = 8


def kernel(x, t_emb, W_scale, W_shift):
    b, s, c_per = x.shape
    sc = s // C

    def body(x_hbm, t_ref, ws_ref, wsh_ref, out_hbm,
             xv, ov, in_sems, out_sems):
        in_dmas = []
        for i in range(C):
            dma = pltpu.make_async_copy(
                x_hbm.at[:, pl.ds(i * sc, sc), :],
                xv.at[:, pl.ds(i * sc, sc), :],
                in_sems.at[i],
            )
            dma.start()
            in_dmas.append(dma)

        cg = 4 * c_per
        scale = jnp.dot(t_ref[...], ws_ref[...],
                        preferred_element_type=jnp.float32)
        shift = jnp.dot(t_ref[...], wsh_ref[...],
                        preferred_element_type=jnp.float32)
        out_dmas = []
        for i in range(C):
            in_dmas[i].wait()
            xs = xv[:, i * sc:(i + 1) * sc, :]
            psum = jnp.sum(xs, axis=-1, keepdims=True)
            psq = jnp.sum(xs * xs, axis=-1, keepdims=True)
            mean = psum * 4.0 / cg
            var = psq * 4.0 / cg - mean * mean
            inv = lax.rsqrt(var + 1e-5)
            h_norm = (xs - mean) * inv
            ov[:, i * sc:(i + 1) * sc, :] = (
                h_norm * (1.0 + scale[:, None, :]) + shift[:, None, :]
            )
            dma = pltpu.make_async_copy(
                ov.at[:, pl.ds(i * sc, sc), :],
                out_hbm.at[:, pl.ds(i * sc, sc), :],
                out_sems.at[i],
            )
            dma.start()
            out_dmas.append(dma)

        for dma in out_dmas:
            dma.wait()

    return pl.pallas_call(
        body,
        out_shape=jax.ShapeDtypeStruct((b, s, c_per), jnp.float32),
        in_specs=[
            pl.BlockSpec(memory_space=pl.ANY),
            pl.BlockSpec(memory_space=pltpu.VMEM),
            pl.BlockSpec(memory_space=pltpu.VMEM),
            pl.BlockSpec(memory_space=pltpu.VMEM),
        ],
        out_specs=pl.BlockSpec(memory_space=pl.ANY),
        scratch_shapes=[
            pltpu.VMEM((b, s, c_per), jnp.float32),
            pltpu.VMEM((b, s, c_per), jnp.float32),
            pltpu.SemaphoreType.DMA((C,)),
            pltpu.SemaphoreType.DMA((C,)),
        ],
    )(x, t_emb, W_scale, W_shift)
